# Initial kernel scaffold; baseline (speedup 1.0000x reference)
import jax
import jax.numpy as jnp
from jax import lax
from jax.experimental import pallas as pl
from jax.experimental.pallas import tpu as pltpu
from jax.experimental.pallas import tpu_sc as plsc

B = 64
L = 4096
DIM = 384
VOCAB = 65

NW = 32
L_PER_W = L // NW
SUB = 32
NSUB = L_PER_W // SUB
NLANE = 16
NCHUNK = DIM // NLANE


def _body(x_hbm, tok_hbm, pos_hbm, out_hbm, table_v, pos_v, idx_v, obuf, sem0, sem1):
    nc = 2
    wid = lax.axis_index("s") * nc + lax.axis_index("c")
    l0 = wid * L_PER_W

    pltpu.sync_copy(tok_hbm, table_v)
    pltpu.sync_copy(pos_hbm.at[pl.ds(l0, L_PER_W), :], pos_v)
    pltpu.sync_copy(x_hbm.at[:, pl.ds(l0, L_PER_W)], idx_v)

    sems = (sem0, sem1)

    def per_batch(b, _):
        for s4 in range(NSUB):
            h = s4 % 2
            @pl.when(jnp.logical_or(b > 0, s4 >= 2))
            def _wait():
                pltpu.make_async_copy(
                    obuf.at[h],
                    out_hbm.at[b, pl.ds(l0 + s4 * SUB, SUB), :],
                    sems[h],
                ).wait()

            def per_token(t, _):
                lt = s4 * SUB + t
                idx = idx_v[b, lt]
                for j in range(NCHUNK):
                    d = pl.ds(j * NLANE, NLANE)
                    obuf[h, t, d] = table_v[idx, d] + pos_v[lt, d]
                return 0

            lax.fori_loop(0, SUB, per_token, 0, unroll=False)
            pltpu.async_copy(
                obuf.at[h],
                out_hbm.at[b, pl.ds(l0 + s4 * SUB, SUB), :],
                sems[h],
            )
        return 0

    lax.fori_loop(0, B, per_batch, 0, unroll=False)

    for s4 in (NSUB - 2, NSUB - 1):
        pltpu.make_async_copy(
            obuf.at[s4 % 2],
            out_hbm.at[B - 1, pl.ds(l0 + s4 * SUB, SUB), :],
            sems[s4 % 2],
        ).wait()


@jax.jit
def kernel(x, token_table, position_table):
    x = x.astype(jnp.int32)
    mesh = plsc.VectorSubcoreMesh(core_axis_name="c", subcore_axis_name="s")
    f = pl.kernel(
        _body,
        out_type=jax.ShapeDtypeStruct((B, L, DIM), jnp.float32),
        mesh=mesh,
        scratch_types=[
            pltpu.VMEM((VOCAB, DIM), jnp.float32),
            pltpu.VMEM((L_PER_W, DIM), jnp.float32),
            pltpu.VMEM((B, L_PER_W), jnp.int32),
            pltpu.VMEM((2, SUB, DIM), jnp.float32),
            pltpu.SemaphoreType.DMA,
            pltpu.SemaphoreType.DMA,
        ],
    )
    return f(x, token_table, position_table)


# speedup vs baseline: 1.2374x; 1.2374x over previous
import jax
import jax.numpy as jnp
from jax import lax
from jax.experimental import pallas as pl
from jax.experimental.pallas import tpu as pltpu
from jax.experimental.pallas import tpu_sc as plsc

B = 64
L = 4096
DIM = 384
VOCAB = 65

NW = 32
L_PER_W = L // NW
SUB = 32
NSUB = L_PER_W // SUB
NLANE = 16
NCHUNK = DIM // NLANE


def _body(x_hbm, tok_hbm, pos_hbm, out_hbm, table_v, pos_v, idx_v, obuf, sem0, sem1):
    nc = 2
    wid = lax.axis_index("s") * nc + lax.axis_index("c")
    l0 = wid * L_PER_W

    pltpu.sync_copy(tok_hbm, table_v)
    pltpu.sync_copy(pos_hbm.at[pl.ds(l0, L_PER_W), :], pos_v)
    pltpu.sync_copy(x_hbm.at[:, pl.ds(l0, L_PER_W)], idx_v)

    sems = (sem0, sem1)

    def per_batch(b, _):
        for s4 in range(NSUB):
            h = s4 % 2
            @pl.when(jnp.logical_or(b > 0, s4 >= 2))
            def _wait():
                pltpu.make_async_copy(
                    obuf.at[h],
                    out_hbm.at[b, pl.ds(l0 + s4 * SUB, SUB), :],
                    sems[h],
                ).wait()

            def per_group(g, _):
                tb = g * NLANE
                ltb = s4 * SUB + tb
                ivec = idx_v[b, pl.ds(ltb, NLANE)]
                idxs = [ivec[k] for k in range(NLANE)]

                def per_dchunk(j, _):
                    d = pl.ds(j * NLANE, NLANE)
                    for k in range(NLANE):
                        obuf[h, tb + k, d] = table_v[idxs[k], d] + pos_v[ltb + k, d]
                    return 0

                lax.fori_loop(0, NCHUNK, per_dchunk, 0, unroll=False)
                return 0

            lax.fori_loop(0, SUB // NLANE, per_group, 0, unroll=False)
            pltpu.async_copy(
                obuf.at[h],
                out_hbm.at[b, pl.ds(l0 + s4 * SUB, SUB), :],
                sems[h],
            )
        return 0

    lax.fori_loop(0, B, per_batch, 0, unroll=False)

    for s4 in (NSUB - 2, NSUB - 1):
        pltpu.make_async_copy(
            obuf.at[s4 % 2],
            out_hbm.at[B - 1, pl.ds(l0 + s4 * SUB, SUB), :],
            sems[s4 % 2],
        ).wait()


@jax.jit
def kernel(x, token_table, position_table):
    x = x.astype(jnp.int32)
    mesh = plsc.VectorSubcoreMesh(core_axis_name="c", subcore_axis_name="s")
    f = pl.kernel(
        _body,
        out_type=jax.ShapeDtypeStruct((B, L, DIM), jnp.float32),
        mesh=mesh,
        scratch_types=[
            pltpu.VMEM((VOCAB, DIM), jnp.float32),
            pltpu.VMEM((L_PER_W, DIM), jnp.float32),
            pltpu.VMEM((B, L_PER_W), jnp.int32),
            pltpu.VMEM((2, SUB, DIM), jnp.float32),
            pltpu.SemaphoreType.DMA,
            pltpu.SemaphoreType.DMA,
        ],
    )
    return f(x, token_table, position_table)
